# baseline (speedup 1.0000x reference)
import jax
import jax.numpy as jnp
from jax.experimental import pallas as pl
from jax.experimental.pallas import tpu as pltpu


def _hgnn_kernel(adj_ref, emb_ref, out_ref, embt_ref):
    h = pl.program_id(0)

    @pl.when(h == 0)
    def _build_embt():
        embt_ref[...] = emb_ref[...].astype(jnp.bfloat16).T

    strip = adj_ref[...].astype(jnp.bfloat16)
    latt = jax.lax.dot_general(
        embt_ref[...], strip, (((1,), (0,)), ((), ())),
        preferred_element_type=jnp.float32)
    lat16 = latt.T.astype(jnp.bfloat16)
    part = jax.lax.dot_general(
        strip, lat16, (((1,), (0,)), ((), ())),
        preferred_element_type=jnp.float32)

    @pl.when(h == 0)
    def _init():
        out_ref[...] = part

    @pl.when(h != 0)
    def _acc():
        out_ref[...] += part


def kernel(adj, embeds):
    n, hh = adj.shape
    d = embeds.shape[1]
    bh = 256
    return pl.pallas_call(
        _hgnn_kernel,
        grid=(hh // bh,),
        in_specs=[
            pl.BlockSpec((n, bh), lambda h: (0, h)),
            pl.BlockSpec((n, d), lambda h: (0, 0)),
        ],
        out_specs=pl.BlockSpec((n, d), lambda h: (0, 0)),
        out_shape=jax.ShapeDtypeStruct((n, d), jnp.float32),
        scratch_shapes=[
            pltpu.VMEM((d, n), jnp.bfloat16),
        ],
    )(adj, embeds)


# speedup vs baseline: 1.1606x; 1.1606x over previous
import jax
import jax.numpy as jnp
from jax.experimental import pallas as pl
from jax.experimental.pallas import tpu as pltpu


def _hgnn_kernel(adja_ref, adjb_ref, emb_ref, out_ref, emb16_ref, acc_ref):
    h = pl.program_id(0)
    nh = pl.num_programs(0)

    @pl.when(h == 0)
    def _cast_emb():
        emb16_ref[...] = emb_ref[...].astype(jnp.bfloat16)

    emb = emb16_ref[...]
    sa = adja_ref[...].astype(jnp.bfloat16)
    sb = adjb_ref[...].astype(jnp.bfloat16)
    lata = jax.lax.dot_general(
        sa, emb, (((0,), (0,)), ((), ())),
        preferred_element_type=jnp.float32).astype(jnp.bfloat16)
    latb = jax.lax.dot_general(
        sb, emb, (((0,), (0,)), ((), ())),
        preferred_element_type=jnp.float32).astype(jnp.bfloat16)
    parta = jax.lax.dot_general(
        sa, lata, (((1,), (0,)), ((), ())),
        preferred_element_type=jnp.float32)
    partb = jax.lax.dot_general(
        sb, latb, (((1,), (0,)), ((), ())),
        preferred_element_type=jnp.float32)
    pair = (parta + partb).astype(jnp.bfloat16)

    @pl.when(h == 0)
    def _init():
        acc_ref[...] = pair

    @pl.when(jnp.logical_and(h != 0, h != nh - 1))
    def _acc():
        acc_ref[...] += pair

    @pl.when(h == nh - 1)
    def _final():
        out_ref[...] = acc_ref[...].astype(jnp.float32) + (parta + partb)


def kernel(adj, embeds):
    n, hh = adj.shape
    d = embeds.shape[1]
    bh = 256
    return pl.pallas_call(
        _hgnn_kernel,
        grid=(hh // (2 * bh),),
        in_specs=[
            pl.BlockSpec((n, bh), lambda h: (0, 2 * h)),
            pl.BlockSpec((n, bh), lambda h: (0, 2 * h + 1)),
            pl.BlockSpec((n, d), lambda h: (0, 0)),
        ],
        out_specs=pl.BlockSpec((n, d), lambda h: (0, 0)),
        out_shape=jax.ShapeDtypeStruct((n, d), jnp.float32),
        scratch_shapes=[
            pltpu.VMEM((n, d), jnp.bfloat16),
            pltpu.VMEM((n, d), jnp.bfloat16),
        ],
        compiler_params=pltpu.CompilerParams(
            vmem_limit_bytes=100 * 1024 * 1024),
    )(adj, adj, embeds)
